# baseline (speedup 1.0000x reference)
import functools

import jax
import jax.numpy as jnp
from jax import lax
from jax.experimental import pallas as pl
from jax.experimental.pallas import tpu as pltpu
from jax.experimental.pallas import tpu_sc as plsc

N_ROWS_K = 16384
CH = 2048
NC = 2
NS = 16
NW = NC * NS
ROWS_PER_W = N_ROWS_K // NW
CHUNK = 16
N_CHUNKS = ROWS_PER_W // CHUNK
LANES = 16
J_GROUPS = CH // LANES


def _body(x_hbm, p_hbm, out_hbm, p_v, inb, outb):
    c = lax.axis_index("c")
    s = lax.axis_index("s")
    wid = s * NC + c
    base = wid * (ROWS_PER_W * CH)

    pltpu.sync_copy(p_hbm, p_v)

    def chunk_body(k, carry):
        e0 = base + k * (CHUNK * CH)
        pltpu.sync_copy(x_hbm.at[pl.ds(e0, CHUNK * CH)], inb)

        def j_body(j, carry2):
            off = pl.multiple_of(j * LANES, LANES)
            idx = p_v[pl.ds(off, LANES)]
            for r in range(CHUNK):
                vals = plsc.load_gather(inb, [idx + jnp.int32(r * CH)])
                outb[pl.ds(off + r * CH, LANES)] = vals
            return carry2

        lax.fori_loop(0, J_GROUPS, j_body, 0, unroll=False)
        pltpu.sync_copy(outb, out_hbm.at[pl.ds(e0, CHUNK * CH)])
        return carry

    lax.fori_loop(0, N_CHUNKS, chunk_body, 0, unroll=False)


@functools.partial(
    pl.kernel,
    out_type=jax.ShapeDtypeStruct((N_ROWS_K * CH,), jnp.float32),
    mesh=plsc.VectorSubcoreMesh(
        core_axis_name="c", subcore_axis_name="s", num_cores=NC, num_subcores=NS
    ),
    scratch_types=[
        pltpu.VMEM((CH,), jnp.int32),
        pltpu.VMEM((CHUNK * CH,), jnp.float32),
        pltpu.VMEM((CHUNK * CH,), jnp.float32),
    ],
    compiler_params=pltpu.CompilerParams(needs_layout_passes=False),
)
def _permute_sc(x_hbm, p_hbm, out_hbm, p_v, inb, outb):
    _body(x_hbm, p_hbm, out_hbm, p_v, inb, outb)


def kernel(x, p):
    out = _permute_sc(x.reshape(-1), p.astype(jnp.int32))
    return (out.reshape(N_ROWS_K, CH), 0)


# speedup vs baseline: 1.1552x; 1.1552x over previous
import functools

import jax
import jax.numpy as jnp
from jax import lax
from jax.experimental import pallas as pl
from jax.experimental.pallas import tpu as pltpu
from jax.experimental.pallas import tpu_sc as plsc

N_ROWS_K = 16384
CH = 2048
NC = 2
NS = 16
NW = NC * NS
ROWS_PER_W = N_ROWS_K // NW
CHUNK = 8
N_CHUNKS = ROWS_PER_W // CHUNK
LANES = 16
J_GROUPS = CH // LANES
CHUNK_E = CHUNK * CH


def _body(x_hbm, p_hbm, out_hbm, p_v, inb0, inb1, outb0, outb1,
          si0, si1, so0, so1):
    c = lax.axis_index("c")
    s = lax.axis_index("s")
    wid = s * NC + c
    base = wid * (ROWS_PER_W * CH)

    inbs = (inb0, inb1)
    outbs = (outb0, outb1)
    sis = (si0, si1)
    sos = (so0, so1)

    def in_slice(q):
        return x_hbm.at[pl.ds(base + q * CHUNK_E, CHUNK_E)]

    def out_slice(q):
        return out_hbm.at[pl.ds(base + q * CHUNK_E, CHUNK_E)]

    pltpu.sync_copy(p_hbm, p_v)

    pltpu.async_copy(in_slice(0), inbs[0], sis[0])

    def do_chunk(q, b):
        @pl.when(q + 1 < N_CHUNKS)
        def _():
            pltpu.async_copy(in_slice(q + 1), inbs[1 - b], sis[1 - b])

        pltpu.make_async_copy(in_slice(q), inbs[b], sis[b]).wait()

        @pl.when(q >= 2)
        def _():
            pltpu.make_async_copy(outbs[b], out_slice(q - 2), sos[b]).wait()

        def j_body(j, carry):
            off = pl.multiple_of(j * LANES, LANES)
            idx = p_v[pl.ds(off, LANES)]
            for r in range(CHUNK):
                vals = plsc.load_gather(inbs[b], [idx + jnp.int32(r * CH)])
                outbs[b][pl.ds(off + r * CH, LANES)] = vals
            return carry

        lax.fori_loop(0, J_GROUPS, j_body, 0, unroll=2)

        pltpu.async_copy(outbs[b], out_slice(q), sos[b])

    @pl.loop(0, N_CHUNKS, step=2)
    def _(k):
        do_chunk(k, 0)
        do_chunk(k + 1, 1)

    pltpu.make_async_copy(outbs[0], out_slice(N_CHUNKS - 2), sos[0]).wait()
    pltpu.make_async_copy(outbs[1], out_slice(N_CHUNKS - 1), sos[1]).wait()


@functools.partial(
    pl.kernel,
    out_type=jax.ShapeDtypeStruct((N_ROWS_K * CH,), jnp.float32),
    mesh=plsc.VectorSubcoreMesh(
        core_axis_name="c", subcore_axis_name="s", num_cores=NC, num_subcores=NS
    ),
    scratch_types=[
        pltpu.VMEM((CH,), jnp.int32),
        pltpu.VMEM((CHUNK_E,), jnp.float32),
        pltpu.VMEM((CHUNK_E,), jnp.float32),
        pltpu.VMEM((CHUNK_E,), jnp.float32),
        pltpu.VMEM((CHUNK_E,), jnp.float32),
        pltpu.SemaphoreType.DMA,
        pltpu.SemaphoreType.DMA,
        pltpu.SemaphoreType.DMA,
        pltpu.SemaphoreType.DMA,
    ],
    compiler_params=pltpu.CompilerParams(needs_layout_passes=False),
)
def _permute_sc(x_hbm, p_hbm, out_hbm, p_v, inb0, inb1, outb0, outb1,
                si0, si1, so0, so1):
    _body(x_hbm, p_hbm, out_hbm, p_v, inb0, inb1, outb0, outb1,
          si0, si1, so0, so1)


def kernel(x, p):
    out = _permute_sc(x.reshape(-1), p.astype(jnp.int32))
    return (out.reshape(N_ROWS_K, CH), 0)


# speedup vs baseline: 1.8240x; 1.5789x over previous
import functools

import jax
import jax.numpy as jnp
from jax import lax
from jax.experimental import pallas as pl
from jax.experimental.pallas import tpu as pltpu
from jax.experimental.pallas import tpu_sc as plsc

N_ROWS_K = 16384
CH = 2048
NC = 2
NS = 16
NW = NC * NS
ROWS_PER_W = N_ROWS_K // NW
CHUNK = 8
N_CHUNKS = ROWS_PER_W // CHUNK
LANES = 16
J_GROUPS = CH // LANES


def _body(x_hbm, p_hbm, out_hbm, p_v, inb0, inb1, outb0, outb1,
          si0, si1, so0, so1):
    c = lax.axis_index("c")
    s = lax.axis_index("s")
    wid = s * NC + c
    row0 = wid * ROWS_PER_W

    inbs = (inb0, inb1)
    outbs = (outb0, outb1)
    sis = (si0, si1)
    sos = (so0, so1)

    def in_slice(q):
        return x_hbm.at[pl.ds(row0 + q * CHUNK, CHUNK), :]

    def out_slice(q):
        return out_hbm.at[pl.ds(row0 + q * CHUNK, CHUNK), :]

    pltpu.sync_copy(p_hbm, p_v)

    pltpu.async_copy(in_slice(0), inbs[0], sis[0])

    def do_chunk(q, b):
        @pl.when(q + 1 < N_CHUNKS)
        def _():
            pltpu.async_copy(in_slice(q + 1), inbs[1 - b], sis[1 - b])

        pltpu.make_async_copy(in_slice(q), inbs[b], sis[b]).wait()

        @pl.when(q >= 2)
        def _():
            pltpu.make_async_copy(outbs[b], out_slice(q - 2), sos[b]).wait()

        def j_body(j, carry):
            off = pl.multiple_of(j * LANES, LANES)
            idx = p_v[pl.ds(off, LANES)]
            for r in range(CHUNK):
                row_idx = jnp.full((LANES,), r, dtype=jnp.int32)
                vals = plsc.load_gather(inbs[b], [row_idx, idx])
                outbs[b][r, pl.ds(off, LANES)] = vals
            return carry

        lax.fori_loop(0, J_GROUPS, j_body, 0, unroll=2)

        pltpu.async_copy(outbs[b], out_slice(q), sos[b])

    @pl.loop(0, N_CHUNKS, step=2)
    def _(k):
        do_chunk(k, 0)
        do_chunk(k + 1, 1)

    pltpu.make_async_copy(outbs[0], out_slice(N_CHUNKS - 2), sos[0]).wait()
    pltpu.make_async_copy(outbs[1], out_slice(N_CHUNKS - 1), sos[1]).wait()


@functools.partial(
    pl.kernel,
    out_type=jax.ShapeDtypeStruct((N_ROWS_K, CH), jnp.float32),
    mesh=plsc.VectorSubcoreMesh(
        core_axis_name="c", subcore_axis_name="s", num_cores=NC, num_subcores=NS
    ),
    scratch_types=[
        pltpu.VMEM((CH,), jnp.int32),
        pltpu.VMEM((CHUNK, CH), jnp.float32),
        pltpu.VMEM((CHUNK, CH), jnp.float32),
        pltpu.VMEM((CHUNK, CH), jnp.float32),
        pltpu.VMEM((CHUNK, CH), jnp.float32),
        pltpu.SemaphoreType.DMA,
        pltpu.SemaphoreType.DMA,
        pltpu.SemaphoreType.DMA,
        pltpu.SemaphoreType.DMA,
    ],
    compiler_params=pltpu.CompilerParams(
        needs_layout_passes=False, use_tc_tiling_on_sc=True
    ),
)
def _permute_sc(x_hbm, p_hbm, out_hbm, p_v, inb0, inb1, outb0, outb1,
                si0, si1, so0, so1):
    _body(x_hbm, p_hbm, out_hbm, p_v, inb0, inb1, outb0, outb1,
          si0, si1, so0, so1)


def kernel(x, p):
    out = _permute_sc(x, p.astype(jnp.int32))
    return (out, 0)
